# Initial kernel scaffold; baseline (speedup 1.0000x reference)
import jax
import jax.numpy as jnp
from jax.experimental import pallas as pl


def kernel(x, adj, num_graphs, in_batch, cluster, gamma, beta, W0, b0, W1, b1, W2, b2, W3, b3, Wc):
    raise NotImplementedError("write your pallas kernel here")


# speedup vs baseline: 7.6347x; 7.6347x over previous
import jax
import jax.numpy as jnp
import numpy as np
from jax import lax
from jax.experimental import pallas as pl
from jax.experimental.pallas import tpu as pltpu
from jax.experimental.pallas import tpu_sc as plsc

_N = 10000
_F = 128
_FH = _F // 2
_E = 320000
_NC = 2
_NS = 16
_ET = _E // _NS
_C = 80
_NCH = _ET // _C
_NP = 10240
_RPT = _NP // _NS
_ZR = 128

_ALPHA = 0.1
_THETA = 0.5
_NUM_LAYERS = 4


def _elu(v):
    return jnp.where(v > 0, v, jnp.exp(v) - 1.0)


def _spmm_body(x_hbm, src_hbm, dst_hbm, out_hbm, sidx, didx, gbuf, zbuf, hsh,
               gsem):
    c = lax.axis_index("c")
    s = lax.axis_index("s")

    pltpu.sync_copy(src_hbm.at[s], sidx)
    pltpu.sync_copy(dst_hbm.at[s], didx)

    zero = jnp.zeros((16,), jnp.float32)

    def zf(i, carry):
        r = i // (_FH // 16)
        k = i % (_FH // 16)
        zbuf[r, pl.ds(k * 16, 16)] = zero
        return carry

    lax.fori_loop(0, _ZR * (_FH // 16), zf, 0)
    rb = s * _RPT
    for j in range(_RPT // _ZR):
        pltpu.sync_copy(zbuf, hsh.at[pl.ds(rb + j * _ZR, _ZR)])
    plsc.subcore_barrier()

    for b in range(2):
        pltpu.async_copy(x_hbm.at[c].at[sidx.at[b]], gbuf.at[b], gsem.at[b])

    def chunk(i, carry):
        for b in range(2):
            cur = 2 * i + b
            pltpu.make_async_copy(
                x_hbm.at[c].at[sidx.at[cur]], gbuf.at[b], gsem.at[b]).wait()
            pltpu.sync_copy(gbuf.at[b], hsh.at[didx.at[cur]], add=True)
            nxt = cur + 2

            @pl.when(nxt < _NCH)
            def _():
                pltpu.async_copy(
                    x_hbm.at[c].at[sidx.at[nxt]], gbuf.at[b], gsem.at[b])
        return carry

    lax.fori_loop(0, _NCH // 2, chunk, 0)

    plsc.subcore_barrier()
    pltpu.sync_copy(hsh.at[pl.ds(rb, _RPT)], out_hbm.at[c, pl.ds(rb, _RPT)])


def _spmm(x_split, src2, dst2):
    mesh = plsc.VectorSubcoreMesh(core_axis_name="c", subcore_axis_name="s")
    f = pl.kernel(
        _spmm_body,
        out_type=jax.ShapeDtypeStruct((_NC, _NP, _FH), jnp.float32),
        mesh=mesh,
        scratch_types=[
            pltpu.VMEM((_NCH, _C), jnp.int32),
            pltpu.VMEM((_NCH, _C), jnp.int32),
            pltpu.VMEM((2, _C, _FH), jnp.float32),
            pltpu.VMEM((_ZR, _FH), jnp.float32),
            pltpu.VMEM_SHARED((_NP, _FH), jnp.float32),
            pltpu.SemaphoreType.DMA((2,)),
        ],
        compiler_params=pltpu.CompilerParams(use_tc_tiling_on_sc=False),
    )
    return f(x_split, src2, dst2)


def _pre_body(x_ref, g_ref, be_ref, w0_ref, b0_ref, w1_ref, b1_ref, o_ref,
              os_ref):
    xv = x_ref[...]
    mean = jnp.mean(xv, axis=0, keepdims=True)
    var = jnp.mean((xv - mean) ** 2, axis=0, keepdims=True)
    xn = (xv - mean) * lax.rsqrt(var + 1e-5)
    xn = xn * g_ref[...] + be_ref[...]
    h = _elu(jnp.dot(xn, w0_ref[...], preferred_element_type=jnp.float32)
             + b0_ref[...])
    h = _elu(jnp.dot(h, w1_ref[...], preferred_element_type=jnp.float32)
             + b1_ref[...])
    o_ref[...] = h
    os_ref[0] = h[:, :_FH]
    os_ref[1] = h[:, _FH:]


def _combine_body(h_ref, x0_ref, w_ref, o_ref, os_ref):
    hsum = jnp.concatenate([h_ref[0, :_N, :], h_ref[1, :_N, :]], axis=1)
    h = (1.0 - _ALPHA) * hsum + _ALPHA * x0_ref[...]
    xn = _elu(jnp.dot(h, w_ref[...], preferred_element_type=jnp.float32))
    o_ref[...] = xn
    os_ref[0] = xn[:, :_FH]
    os_ref[1] = xn[:, _FH:]


def _post_body(x_ref, w2_ref, b2_ref, w3_ref, b3_ref, o_ref):
    h = _elu(jnp.dot(x_ref[...], w2_ref[...],
                     preferred_element_type=jnp.float32) + b2_ref[...])
    o_ref[...] = _elu(jnp.dot(h, w3_ref[...],
                              preferred_element_type=jnp.float32)
                      + b3_ref[...])


def kernel(x, adj, num_graphs, in_batch, cluster, gamma, beta, W0, b0, W1,
           b1, W2, b2, W3, b3, Wc):
    h2 = W1.shape[1]
    out_dim = W3.shape[1]

    src2 = adj[0].reshape(_NS, _NCH, _C)
    dst2 = adj[1].reshape(_NS, _NCH, _C)

    x0, x0s = pl.pallas_call(
        _pre_body,
        out_shape=(
            jax.ShapeDtypeStruct((_N, h2), jnp.float32),
            jax.ShapeDtypeStruct((_NC, _N, _FH), jnp.float32),
        ),
    )(x, gamma.reshape(1, -1), beta.reshape(1, -1), W0, b0.reshape(1, -1),
      W1, b1.reshape(1, -1))

    combine = pl.pallas_call(
        _combine_body,
        out_shape=(
            jax.ShapeDtypeStruct((_N, h2), jnp.float32),
            jax.ShapeDtypeStruct((_NC, _N, _FH), jnp.float32),
        ),
    )

    eye = jnp.eye(h2, dtype=jnp.float32)
    xc, xcs = x0, x0s
    for layer in range(_NUM_LAYERS):
        beta_l = float(np.log(_THETA / (layer + 1) + 1.0))
        w_eff = (1.0 - beta_l) * eye + beta_l * Wc[layer]
        hparts = _spmm(xcs, src2, dst2)
        xc, xcs = combine(hparts, x0, w_eff)

    return pl.pallas_call(
        _post_body,
        out_shape=jax.ShapeDtypeStruct((_N, out_dim), jnp.float32),
    )(xc, W2, b2.reshape(1, -1), W3, b3.reshape(1, -1))


# speedup vs baseline: 9.2349x; 1.2096x over previous
import jax
import jax.numpy as jnp
import numpy as np
from jax import lax
from jax.experimental import pallas as pl
from jax.experimental.pallas import tpu as pltpu
from jax.experimental.pallas import tpu_sc as plsc

_N = 10000
_F = 128
_FH = _F // 2
_E = 320000
_NC = 2
_NS = 16
_ET = _E // _NS
_C = 125
_NCH = _ET // _C
_NB = 4
_NP = 10240
_RPT = _NP // _NS
_ZR = 128

_ALPHA = 0.1
_THETA = 0.5
_NUM_LAYERS = 4


def _elu(v):
    return jnp.where(v > 0, v, jnp.exp(v) - 1.0)


def _spmm_body(x_hbm, src_hbm, dst_hbm, out_hbm, sidx, didx, gbuf, zbuf, hsh,
               gsem, ssem):
    c = lax.axis_index("c")
    s = lax.axis_index("s")

    pltpu.sync_copy(src_hbm.at[s], sidx)
    pltpu.sync_copy(dst_hbm.at[s], didx)

    zero = jnp.zeros((16,), jnp.float32)

    def zf(i, carry):
        r = i // (_FH // 16)
        k = i % (_FH // 16)
        zbuf[r, pl.ds(k * 16, 16)] = zero
        return carry

    lax.fori_loop(0, _ZR * (_FH // 16), zf, 0)
    rb = s * _RPT
    for j in range(_RPT // _ZR):
        pltpu.sync_copy(zbuf, hsh.at[pl.ds(rb + j * _ZR, _ZR)])
    plsc.subcore_barrier()

    for b in range(2):
        pltpu.async_copy(x_hbm.at[c].at[sidx.at[b]], gbuf.at[b], gsem.at[b])

    def chunk(i, carry):
        for b in range(_NB):
            cur = _NB * i + b
            bn = (b + 2) % _NB
            pltpu.make_async_copy(
                x_hbm.at[c].at[sidx.at[cur]], gbuf.at[b], gsem.at[b]).wait()
            pltpu.async_copy(gbuf.at[b], hsh.at[didx.at[cur]], ssem.at[b],
                             add=True)
            nxt = cur + 2

            @pl.when(nxt < _NCH)
            def _():
                @pl.when(nxt >= _NB)
                def _():
                    pltpu.make_async_copy(
                        gbuf.at[bn], hsh.at[didx.at[cur]],
                        ssem.at[bn]).wait()

                pltpu.async_copy(
                    x_hbm.at[c].at[sidx.at[nxt]], gbuf.at[bn], gsem.at[bn])
        return carry

    lax.fori_loop(0, _NCH // _NB, chunk, 0)
    for j in range(_NB):
        b = (_NCH - _NB + j) % _NB
        pltpu.make_async_copy(
            gbuf.at[b], hsh.at[didx.at[_NCH - _NB + j]], ssem.at[b]).wait()

    plsc.subcore_barrier()
    pltpu.sync_copy(hsh.at[pl.ds(rb, _RPT)], out_hbm.at[c, pl.ds(rb, _RPT)])


def _spmm(x_split, src2, dst2):
    mesh = plsc.VectorSubcoreMesh(core_axis_name="c", subcore_axis_name="s")
    f = pl.kernel(
        _spmm_body,
        out_type=jax.ShapeDtypeStruct((_NC, _NP, _FH), jnp.float32),
        mesh=mesh,
        scratch_types=[
            pltpu.VMEM((_NCH, _C), jnp.int32),
            pltpu.VMEM((_NCH, _C), jnp.int32),
            pltpu.VMEM((_NB, _C, _FH), jnp.float32),
            pltpu.VMEM((_ZR, _FH), jnp.float32),
            pltpu.VMEM_SHARED((_NP, _FH), jnp.float32),
            pltpu.SemaphoreType.DMA((_NB,)),
            pltpu.SemaphoreType.DMA((_NB,)),
        ],
        compiler_params=pltpu.CompilerParams(use_tc_tiling_on_sc=False),
    )
    return f(x_split, src2, dst2)


def _pre_body(x_ref, g_ref, be_ref, w0_ref, b0_ref, w1_ref, b1_ref, o_ref,
              os_ref):
    xv = x_ref[...]
    mean = jnp.mean(xv, axis=0, keepdims=True)
    var = jnp.mean((xv - mean) ** 2, axis=0, keepdims=True)
    xn = (xv - mean) * lax.rsqrt(var + 1e-5)
    xn = xn * g_ref[...] + be_ref[...]
    h = _elu(jnp.dot(xn, w0_ref[...], preferred_element_type=jnp.float32)
             + b0_ref[...])
    h = _elu(jnp.dot(h, w1_ref[...], preferred_element_type=jnp.float32)
             + b1_ref[...])
    o_ref[...] = h
    os_ref[0] = h[:, :_FH]
    os_ref[1] = h[:, _FH:]


def _combine_body(h_ref, x0_ref, w_ref, o_ref, os_ref):
    hsum = jnp.concatenate([h_ref[0, :_N, :], h_ref[1, :_N, :]], axis=1)
    h = (1.0 - _ALPHA) * hsum + _ALPHA * x0_ref[...]
    xn = _elu(jnp.dot(h, w_ref[...], preferred_element_type=jnp.float32))
    o_ref[...] = xn
    os_ref[0] = xn[:, :_FH]
    os_ref[1] = xn[:, _FH:]


def _post_body(x_ref, w2_ref, b2_ref, w3_ref, b3_ref, o_ref):
    h = _elu(jnp.dot(x_ref[...], w2_ref[...],
                     preferred_element_type=jnp.float32) + b2_ref[...])
    o_ref[...] = _elu(jnp.dot(h, w3_ref[...],
                              preferred_element_type=jnp.float32)
                      + b3_ref[...])


def kernel(x, adj, num_graphs, in_batch, cluster, gamma, beta, W0, b0, W1,
           b1, W2, b2, W3, b3, Wc):
    h2 = W1.shape[1]
    out_dim = W3.shape[1]

    src2 = adj[0].reshape(_NS, _NCH, _C)
    dst2 = adj[1].reshape(_NS, _NCH, _C)

    x0, x0s = pl.pallas_call(
        _pre_body,
        out_shape=(
            jax.ShapeDtypeStruct((_N, h2), jnp.float32),
            jax.ShapeDtypeStruct((_NC, _N, _FH), jnp.float32),
        ),
    )(x, gamma.reshape(1, -1), beta.reshape(1, -1), W0, b0.reshape(1, -1),
      W1, b1.reshape(1, -1))

    combine = pl.pallas_call(
        _combine_body,
        out_shape=(
            jax.ShapeDtypeStruct((_N, h2), jnp.float32),
            jax.ShapeDtypeStruct((_NC, _N, _FH), jnp.float32),
        ),
    )

    eye = jnp.eye(h2, dtype=jnp.float32)
    xc, xcs = x0, x0s
    for layer in range(_NUM_LAYERS):
        beta_l = float(np.log(_THETA / (layer + 1) + 1.0))
        w_eff = (1.0 - beta_l) * eye + beta_l * Wc[layer]
        hparts = _spmm(xcs, src2, dst2)
        xc, xcs = combine(hparts, x0, w_eff)

    return pl.pallas_call(
        _post_body,
        out_shape=jax.ShapeDtypeStruct((_N, out_dim), jnp.float32),
    )(xc, W2, b2.reshape(1, -1), W3, b3.reshape(1, -1))


# speedup vs baseline: 9.3628x; 1.0139x over previous
import jax
import jax.numpy as jnp
import numpy as np
from jax import lax
from jax.experimental import pallas as pl
from jax.experimental.pallas import tpu as pltpu
from jax.experimental.pallas import tpu_sc as plsc

_N = 10000
_F = 128
_FH = _F // 2
_E = 320000
_NC = 2
_NS = 16
_ET = _E // _NS
_C = 80
_NCH = _ET // _C
_NB = 5
_GA = 2
_NI = 12
_IA = 6
_NP = 10240
_RPT = _NP // _NS

_ALPHA = 0.1
_THETA = 0.5
_NUM_LAYERS = 4


def _elu(v):
    return jnp.where(v > 0, v, jnp.exp(v) - 1.0)


def _spmm_body(x_hbm, adj_hbm, out_hbm, ibuf, gbuf, hsh, xsh,
               isem, gsem, ssem):
    c = lax.axis_index("c")
    s = lax.axis_index("s")

    for j in range(_IA):
        pltpu.async_copy(adj_hbm.at[s, j], ibuf.at[j], isem.at[j])

    @pl.when(s < 10)
    def _():
        pltpu.sync_copy(x_hbm.at[c, pl.ds(s * 1000, 1000)],
                        xsh.at[pl.ds(s * 1000, 1000)])

    zero = jnp.zeros((16,), jnp.float32)

    def zf(i, carry):
        r = i // (_FH // 16)
        k = i % (_FH // 16)
        gbuf[0, r, pl.ds(k * 16, 16)] = zero
        return carry

    lax.fori_loop(0, _C * (_FH // 16), zf, 0)
    rb = s * _RPT
    for j in range(_RPT // _C):
        pltpu.sync_copy(gbuf.at[0], hsh.at[pl.ds(rb + j * _C, _C)])
    plsc.subcore_barrier()

    for b in range(_GA):
        pltpu.make_async_copy(adj_hbm.at[s, b], ibuf.at[b], isem.at[b]).wait()
        pltpu.async_copy(xsh.at[ibuf.at[b, 0]], gbuf.at[b],
                         gsem.at[b])

    def chunk(i, carry):
        for b in range(_NB):
            cur = _NB * i + b
            bn = (b + _GA) % _NB
            pltpu.make_async_copy(
                xsh.at[ibuf.at[b, 0]], gbuf.at[b], gsem.at[b]).wait()
            pltpu.async_copy(gbuf.at[b], hsh.at[ibuf.at[cur % _NI, 1]],
                             ssem.at[b], add=True)
            nxt = cur + _GA

            @pl.when(nxt < _NCH)
            def _():
                @pl.when(nxt >= _NB)
                def _():
                    pltpu.make_async_copy(
                        gbuf.at[bn], hsh.at[ibuf.at[0, 1]],
                        ssem.at[bn]).wait()

                pltpu.make_async_copy(
                    adj_hbm.at[s, nxt], ibuf.at[nxt % _NI],
                    isem.at[nxt % _NI]).wait()
                pltpu.async_copy(xsh.at[ibuf.at[nxt % _NI, 0]],
                                 gbuf.at[bn], gsem.at[bn])

            nf = cur + _IA

            @pl.when(nf < _NCH)
            def _():
                pltpu.async_copy(adj_hbm.at[s, nf], ibuf.at[nf % _NI],
                                 isem.at[nf % _NI])
        return carry

    lax.fori_loop(0, _NCH // _NB, chunk, 0)
    for j in range(_NB):
        b = (_NCH - _NB + j) % _NB
        pltpu.make_async_copy(
            gbuf.at[b], hsh.at[ibuf.at[0, 1]], ssem.at[b]).wait()

    plsc.subcore_barrier()
    pltpu.sync_copy(hsh.at[pl.ds(rb, _RPT)], out_hbm.at[c, pl.ds(rb, _RPT)])


def _spmm(x_split, adj4):
    mesh = plsc.VectorSubcoreMesh(core_axis_name="c", subcore_axis_name="s")
    f = pl.kernel(
        _spmm_body,
        out_type=jax.ShapeDtypeStruct((_NC, _NP, _FH), jnp.float32),
        mesh=mesh,
        scratch_types=[
            pltpu.VMEM((_NI, 2, _C), jnp.int32),
            pltpu.VMEM((_NB, _C, _FH), jnp.float32),
            pltpu.VMEM_SHARED((_NP, _FH), jnp.float32),
            pltpu.VMEM_SHARED((_N, _FH), jnp.float32),
            pltpu.SemaphoreType.DMA((_NI,)),
            pltpu.SemaphoreType.DMA((_NB,)),
            pltpu.SemaphoreType.DMA((_NB,)),
        ],
        compiler_params=pltpu.CompilerParams(use_tc_tiling_on_sc=False),
    )
    return f(x_split, adj4)


def _pre_body(x_ref, g_ref, be_ref, w0_ref, b0_ref, w1_ref, b1_ref, o_ref,
              os_ref):
    xv = x_ref[...]
    mean = jnp.mean(xv, axis=0, keepdims=True)
    var = jnp.mean((xv - mean) ** 2, axis=0, keepdims=True)
    xn = (xv - mean) * lax.rsqrt(var + 1e-5)
    xn = xn * g_ref[...] + be_ref[...]
    h = _elu(jnp.dot(xn, w0_ref[...], preferred_element_type=jnp.float32)
             + b0_ref[...])
    h = _elu(jnp.dot(h, w1_ref[...], preferred_element_type=jnp.float32)
             + b1_ref[...])
    o_ref[...] = h
    os_ref[0] = h[:, :_FH]
    os_ref[1] = h[:, _FH:]


def _combine_body(h_ref, x0_ref, w_ref, o_ref, os_ref):
    hsum = jnp.concatenate([h_ref[0, :_N, :], h_ref[1, :_N, :]], axis=1)
    h = (1.0 - _ALPHA) * hsum + _ALPHA * x0_ref[...]
    xn = _elu(jnp.dot(h, w_ref[...], preferred_element_type=jnp.float32))
    o_ref[...] = xn
    os_ref[0] = xn[:, :_FH]
    os_ref[1] = xn[:, _FH:]


def _post_body(x_ref, w2_ref, b2_ref, w3_ref, b3_ref, o_ref):
    h = _elu(jnp.dot(x_ref[...], w2_ref[...],
                     preferred_element_type=jnp.float32) + b2_ref[...])
    o_ref[...] = _elu(jnp.dot(h, w3_ref[...],
                              preferred_element_type=jnp.float32)
                      + b3_ref[...])


def kernel(x, adj, num_graphs, in_batch, cluster, gamma, beta, W0, b0, W1,
           b1, W2, b2, W3, b3, Wc):
    h2 = W1.shape[1]
    out_dim = W3.shape[1]

    adj4 = adj.reshape(2, _NS, _NCH, _C).transpose(1, 2, 0, 3)

    x0, x0s = pl.pallas_call(
        _pre_body,
        out_shape=(
            jax.ShapeDtypeStruct((_N, h2), jnp.float32),
            jax.ShapeDtypeStruct((_NC, _N, _FH), jnp.float32),
        ),
    )(x, gamma.reshape(1, -1), beta.reshape(1, -1), W0, b0.reshape(1, -1),
      W1, b1.reshape(1, -1))

    combine = pl.pallas_call(
        _combine_body,
        out_shape=(
            jax.ShapeDtypeStruct((_N, h2), jnp.float32),
            jax.ShapeDtypeStruct((_NC, _N, _FH), jnp.float32),
        ),
    )

    eye = jnp.eye(h2, dtype=jnp.float32)
    xc, xcs = x0, x0s
    for layer in range(_NUM_LAYERS):
        beta_l = float(np.log(_THETA / (layer + 1) + 1.0))
        w_eff = (1.0 - beta_l) * eye + beta_l * Wc[layer]
        hparts = _spmm(xcs, adj4)
        xc, xcs = combine(hparts, x0, w_eff)

    return pl.pallas_call(
        _post_body,
        out_shape=jax.ShapeDtypeStruct((_N, out_dim), jnp.float32),
    )(xc, W2, b2.reshape(1, -1), W3, b3.reshape(1, -1))


# speedup vs baseline: 9.3895x; 1.0028x over previous
import jax
import jax.numpy as jnp
import numpy as np
from jax import lax
from jax.experimental import pallas as pl
from jax.experimental.pallas import tpu as pltpu
from jax.experimental.pallas import tpu_sc as plsc

_N = 10000
_F = 128
_FH = _F // 2
_E = 320000
_NC = 2
_NS = 16
_ET = _E // _NS
_C = 80
_NCH = _ET // _C
_NB = 5
_GA = 3
_NI = 12
_IA = 6
_NP = 10240
_RPT = _NP // _NS

_ALPHA = 0.1
_THETA = 0.5
_NUM_LAYERS = 4


def _elu(v):
    return jnp.where(v > 0, v, jnp.exp(v) - 1.0)


def _spmm_body(x_hbm, adj_hbm, out_hbm, ibuf, gbuf, hsh, xsh,
               isem, gsem, ssem):
    c = lax.axis_index("c")
    s = lax.axis_index("s")

    for j in range(_IA):
        pltpu.async_copy(adj_hbm.at[s, j], ibuf.at[j], isem.at[j])

    @pl.when(s < 10)
    def _():
        pltpu.sync_copy(x_hbm.at[c, pl.ds(s * 1000, 1000)],
                        xsh.at[pl.ds(s * 1000, 1000)])

    zero = jnp.zeros((16,), jnp.float32)

    def zf(i, carry):
        r = i // (_FH // 16)
        k = i % (_FH // 16)
        gbuf[0, r, pl.ds(k * 16, 16)] = zero
        return carry

    lax.fori_loop(0, _C * (_FH // 16), zf, 0)
    rb = s * _RPT
    for j in range(_RPT // _C):
        pltpu.sync_copy(gbuf.at[0], hsh.at[pl.ds(rb + j * _C, _C)])
    plsc.subcore_barrier()

    for b in range(_GA):
        pltpu.make_async_copy(adj_hbm.at[s, b], ibuf.at[b], isem.at[b]).wait()
        pltpu.async_copy(xsh.at[ibuf.at[b, 0]], gbuf.at[b],
                         gsem.at[b])

    def chunk(i, carry):
        for b in range(_NB):
            cur = _NB * i + b
            bn = (b + _GA) % _NB
            pltpu.make_async_copy(
                xsh.at[ibuf.at[b, 0]], gbuf.at[b], gsem.at[b]).wait()
            pltpu.async_copy(gbuf.at[b], hsh.at[ibuf.at[cur % _NI, 1]],
                             ssem.at[b], add=True)
            nxt = cur + _GA

            @pl.when(nxt < _NCH)
            def _():
                @pl.when(nxt >= _NB)
                def _():
                    pltpu.make_async_copy(
                        gbuf.at[bn], hsh.at[ibuf.at[0, 1]],
                        ssem.at[bn]).wait()

                pltpu.make_async_copy(
                    adj_hbm.at[s, nxt], ibuf.at[nxt % _NI],
                    isem.at[nxt % _NI]).wait()
                pltpu.async_copy(xsh.at[ibuf.at[nxt % _NI, 0]],
                                 gbuf.at[bn], gsem.at[bn])

            nf = cur + _IA

            @pl.when(nf < _NCH)
            def _():
                pltpu.async_copy(adj_hbm.at[s, nf], ibuf.at[nf % _NI],
                                 isem.at[nf % _NI])
        return carry

    lax.fori_loop(0, _NCH // _NB, chunk, 0)
    for j in range(_NB):
        b = (_NCH - _NB + j) % _NB
        pltpu.make_async_copy(
            gbuf.at[b], hsh.at[ibuf.at[0, 1]], ssem.at[b]).wait()

    plsc.subcore_barrier()
    pltpu.sync_copy(hsh.at[pl.ds(rb, _RPT)], out_hbm.at[c, pl.ds(rb, _RPT)])


def _spmm(x_split, adj4):
    mesh = plsc.VectorSubcoreMesh(core_axis_name="c", subcore_axis_name="s")
    f = pl.kernel(
        _spmm_body,
        out_type=jax.ShapeDtypeStruct((_NC, _NP, _FH), jnp.float32),
        mesh=mesh,
        scratch_types=[
            pltpu.VMEM((_NI, 2, _C), jnp.int32),
            pltpu.VMEM((_NB, _C, _FH), jnp.float32),
            pltpu.VMEM_SHARED((_NP, _FH), jnp.float32),
            pltpu.VMEM_SHARED((_N, _FH), jnp.float32),
            pltpu.SemaphoreType.DMA((_NI,)),
            pltpu.SemaphoreType.DMA((_NB,)),
            pltpu.SemaphoreType.DMA((_NB,)),
        ],
        compiler_params=pltpu.CompilerParams(use_tc_tiling_on_sc=False),
    )
    return f(x_split, adj4)


def _pre_body(x_ref, g_ref, be_ref, w0_ref, b0_ref, w1_ref, b1_ref, o_ref,
              os_ref):
    xv = x_ref[...]
    mean = jnp.mean(xv, axis=0, keepdims=True)
    var = jnp.mean((xv - mean) ** 2, axis=0, keepdims=True)
    xn = (xv - mean) * lax.rsqrt(var + 1e-5)
    xn = xn * g_ref[...] + be_ref[...]
    h = _elu(jnp.dot(xn, w0_ref[...], preferred_element_type=jnp.float32)
             + b0_ref[...])
    h = _elu(jnp.dot(h, w1_ref[...], preferred_element_type=jnp.float32)
             + b1_ref[...])
    o_ref[...] = h
    os_ref[0] = h[:, :_FH]
    os_ref[1] = h[:, _FH:]


def _combine_body(h_ref, x0_ref, w_ref, o_ref, os_ref):
    hsum = jnp.concatenate([h_ref[0, :_N, :], h_ref[1, :_N, :]], axis=1)
    h = (1.0 - _ALPHA) * hsum + _ALPHA * x0_ref[...]
    xn = _elu(jnp.dot(h, w_ref[...], preferred_element_type=jnp.float32))
    o_ref[...] = xn
    os_ref[0] = xn[:, :_FH]
    os_ref[1] = xn[:, _FH:]


def _post_body(x_ref, w2_ref, b2_ref, w3_ref, b3_ref, o_ref):
    h = _elu(jnp.dot(x_ref[...], w2_ref[...],
                     preferred_element_type=jnp.float32) + b2_ref[...])
    o_ref[...] = _elu(jnp.dot(h, w3_ref[...],
                              preferred_element_type=jnp.float32)
                      + b3_ref[...])


def kernel(x, adj, num_graphs, in_batch, cluster, gamma, beta, W0, b0, W1,
           b1, W2, b2, W3, b3, Wc):
    h2 = W1.shape[1]
    out_dim = W3.shape[1]

    adj4 = adj.reshape(2, _NS, _NCH, _C).transpose(1, 2, 0, 3)

    x0, x0s = pl.pallas_call(
        _pre_body,
        out_shape=(
            jax.ShapeDtypeStruct((_N, h2), jnp.float32),
            jax.ShapeDtypeStruct((_NC, _N, _FH), jnp.float32),
        ),
    )(x, gamma.reshape(1, -1), beta.reshape(1, -1), W0, b0.reshape(1, -1),
      W1, b1.reshape(1, -1))

    combine = pl.pallas_call(
        _combine_body,
        out_shape=(
            jax.ShapeDtypeStruct((_N, h2), jnp.float32),
            jax.ShapeDtypeStruct((_NC, _N, _FH), jnp.float32),
        ),
    )

    eye = jnp.eye(h2, dtype=jnp.float32)
    xc, xcs = x0, x0s
    for layer in range(_NUM_LAYERS):
        beta_l = float(np.log(_THETA / (layer + 1) + 1.0))
        w_eff = (1.0 - beta_l) * eye + beta_l * Wc[layer]
        hparts = _spmm(xcs, adj4)
        xc, xcs = combine(hparts, x0, w_eff)

    return pl.pallas_call(
        _post_body,
        out_shape=jax.ShapeDtypeStruct((_N, out_dim), jnp.float32),
    )(xc, W2, b2.reshape(1, -1), W3, b3.reshape(1, -1))
